# baseline (speedup 1.0000x reference)
import functools

import jax
import jax.numpy as jnp
from jax.experimental import pallas as pl
from jax.experimental.pallas import tpu as pltpu

T = 64
D = 1024
F = 1408
E = 16
FT = 128
NF = F // FT
SCALE = 1.0 / (2.0 ** 0.5)
HIGHEST = jax.lax.Precision.HIGHEST


def _dot(a, b):
    return jax.lax.dot_general(a, b, (((1,), (0,)), ((), ())),
                               precision=HIGHEST,
                               preferred_element_type=jnp.float32)


def _moe_body(x_ref, rw_ref, eg_ref, eu_ref, ed_ref, sg_ref, su_ref, sd_ref,
              out_ref, comb_ref):
    f = pl.program_id(0)
    e = pl.program_id(1)

    @pl.when((f == 0) & (e == 0))
    def _init():
        x = x_ref[...]
        lg = jax.lax.dot_general(x, rw_ref[...], (((1,), (1,)), ((), ())),
                                 precision=HIGHEST,
                                 preferred_element_type=jnp.float32)
        logits = lg[:, :E]
        iota = jax.lax.broadcasted_iota(jnp.int32, (T, E), 1)
        v1 = jnp.max(logits, axis=-1, keepdims=True)
        i1 = jnp.min(jnp.where(logits == v1, iota, E), axis=-1, keepdims=True)
        masked = jnp.where(iota == i1, -jnp.inf, logits)
        v2 = jnp.max(masked, axis=-1, keepdims=True)
        i2 = jnp.min(jnp.where(masked == v2, iota, E), axis=-1, keepdims=True)
        w1 = jax.nn.sigmoid(v1 - v2)
        w2 = jax.nn.sigmoid(v2 - v1)
        sscore = jax.nn.sigmoid(lg[:, E:E + 1])
        lane = jax.lax.broadcasted_iota(jnp.int32, (T, 128), 1)
        comb_full = (jnp.where(lane == i1, w1, 0.0)
                     + jnp.where(lane == i2, w2, 0.0)
                     + jnp.where(lane == E, sscore, 0.0)) * SCALE
        comb_ref[...] = comb_full
        out_ref[...] = jnp.zeros_like(out_ref)

    x = x_ref[...]

    def ffn(wg, wu, wd, w):
        g = _dot(x, wg)
        u = _dot(x, wu)
        h = jax.nn.silu(g) * u * w
        return _dot(h, wd)

    lane = jax.lax.broadcasted_iota(jnp.int32, (T, 128), 1)
    comb = comb_ref[...]
    w_e = jnp.sum(jnp.where(lane == e, comb, 0.0), axis=1, keepdims=True)
    out_ref[...] += ffn(eg_ref[0], eu_ref[0], ed_ref[0], w_e)

    @pl.when(e == 0)
    def _shared():
        w_s = jnp.sum(jnp.where(lane == E, comb, 0.0), axis=1, keepdims=True)
        out_ref[...] += ffn(sg_ref[0], su_ref[0], sd_ref[0], w_s)


@functools.partial(jax.jit, static_argnames=())
def kernel(hidden_states, gate_w, shared_expert_gate_w, expert_gate_w,
           expert_up_w, expert_down_w, shared_gate_proj_w, shared_up_w,
           shared_down_w):
    x = hidden_states.reshape(T, D)
    rw = jnp.concatenate(
        [gate_w, shared_expert_gate_w,
         jnp.zeros((7, D), dtype=gate_w.dtype)], axis=0)

    grid = (NF, E)
    out = pl.pallas_call(
        _moe_body,
        grid=grid,
        in_specs=[
            pl.BlockSpec((T, D), lambda f, e: (0, 0)),
            pl.BlockSpec((24, D), lambda f, e: (0, 0)),
            pl.BlockSpec((1, D, FT), lambda f, e: (e, 0, f)),
            pl.BlockSpec((1, D, FT), lambda f, e: (e, 0, f)),
            pl.BlockSpec((1, FT, D), lambda f, e: (e, f, 0)),
            pl.BlockSpec((1, D, FT), lambda f, e: (0, 0, f)),
            pl.BlockSpec((1, D, FT), lambda f, e: (0, 0, f)),
            pl.BlockSpec((1, FT, D), lambda f, e: (0, f, 0)),
        ],
        out_specs=pl.BlockSpec((T, D), lambda f, e: (0, 0)),
        out_shape=jax.ShapeDtypeStruct((T, D), jnp.float32),
        scratch_shapes=[pltpu.VMEM((T, 128), jnp.float32)],
        compiler_params=pltpu.CompilerParams(
            dimension_semantics=("arbitrary", "arbitrary"),
        ),
    )(x, rw, expert_gate_w, expert_up_w, expert_down_w,
      shared_gate_proj_w, shared_up_w, shared_down_w)
    return out


# speedup vs baseline: 1.3183x; 1.3183x over previous
import functools

import jax
import jax.numpy as jnp
from jax.experimental import pallas as pl
from jax.experimental.pallas import tpu as pltpu

T = 64
D = 1024
F = 1408
E = 16
FT = 128
NF = F // FT
SCALE = 1.0 / (2.0 ** 0.5)
HIGHEST = jax.lax.Precision.DEFAULT


def _dot(a, b):
    return jax.lax.dot_general(a, b, (((1,), (0,)), ((), ())),
                               precision=HIGHEST,
                               preferred_element_type=jnp.float32)


def _moe_body(x_ref, rw_ref, eg_ref, eu_ref, ed_ref, sg_ref, su_ref, sd_ref,
              out_ref, comb_ref):
    f = pl.program_id(0)
    e = pl.program_id(1)

    @pl.when((f == 0) & (e == 0))
    def _init():
        x = x_ref[...]
        lg = jax.lax.dot_general(x, rw_ref[...], (((1,), (1,)), ((), ())),
                                 precision=HIGHEST,
                                 preferred_element_type=jnp.float32)
        logits = lg[:, :E]
        iota = jax.lax.broadcasted_iota(jnp.int32, (T, E), 1)
        v1 = jnp.max(logits, axis=-1, keepdims=True)
        i1 = jnp.min(jnp.where(logits == v1, iota, E), axis=-1, keepdims=True)
        masked = jnp.where(iota == i1, -jnp.inf, logits)
        v2 = jnp.max(masked, axis=-1, keepdims=True)
        i2 = jnp.min(jnp.where(masked == v2, iota, E), axis=-1, keepdims=True)
        w1 = jax.nn.sigmoid(v1 - v2)
        w2 = jax.nn.sigmoid(v2 - v1)
        sscore = jax.nn.sigmoid(lg[:, E:E + 1])
        lane = jax.lax.broadcasted_iota(jnp.int32, (T, 128), 1)
        comb_full = (jnp.where(lane == i1, w1, 0.0)
                     + jnp.where(lane == i2, w2, 0.0)
                     + jnp.where(lane == E, sscore, 0.0)) * SCALE
        comb_ref[...] = comb_full
        out_ref[...] = jnp.zeros_like(out_ref)

    x = x_ref[...]

    def ffn(wg, wu, wd, w):
        g = _dot(x, wg)
        u = _dot(x, wu)
        h = jax.nn.silu(g) * u * w
        return _dot(h, wd)

    lane = jax.lax.broadcasted_iota(jnp.int32, (T, 128), 1)
    comb = comb_ref[...]
    w_e = jnp.sum(jnp.where(lane == e, comb, 0.0), axis=1, keepdims=True)
    out_ref[...] += ffn(eg_ref[0], eu_ref[0], ed_ref[0], w_e)

    @pl.when(e == 0)
    def _shared():
        w_s = jnp.sum(jnp.where(lane == E, comb, 0.0), axis=1, keepdims=True)
        out_ref[...] += ffn(sg_ref[0], su_ref[0], sd_ref[0], w_s)


@functools.partial(jax.jit, static_argnames=())
def kernel(hidden_states, gate_w, shared_expert_gate_w, expert_gate_w,
           expert_up_w, expert_down_w, shared_gate_proj_w, shared_up_w,
           shared_down_w):
    x = hidden_states.reshape(T, D)
    rw = jnp.concatenate(
        [gate_w, shared_expert_gate_w,
         jnp.zeros((7, D), dtype=gate_w.dtype)], axis=0)

    grid = (NF, E)
    out = pl.pallas_call(
        _moe_body,
        grid=grid,
        in_specs=[
            pl.BlockSpec((T, D), lambda f, e: (0, 0)),
            pl.BlockSpec((24, D), lambda f, e: (0, 0)),
            pl.BlockSpec((1, D, FT), lambda f, e: (e, 0, f)),
            pl.BlockSpec((1, D, FT), lambda f, e: (e, 0, f)),
            pl.BlockSpec((1, FT, D), lambda f, e: (e, f, 0)),
            pl.BlockSpec((1, D, FT), lambda f, e: (0, 0, f)),
            pl.BlockSpec((1, D, FT), lambda f, e: (0, 0, f)),
            pl.BlockSpec((1, FT, D), lambda f, e: (0, f, 0)),
        ],
        out_specs=pl.BlockSpec((T, D), lambda f, e: (0, 0)),
        out_shape=jax.ShapeDtypeStruct((T, D), jnp.float32),
        scratch_shapes=[pltpu.VMEM((T, 128), jnp.float32)],
        compiler_params=pltpu.CompilerParams(
            dimension_semantics=("arbitrary", "arbitrary"),
        ),
    )(x, rw, expert_gate_w, expert_up_w, expert_down_w,
      shared_gate_proj_w, shared_up_w, shared_down_w)
    return out


# speedup vs baseline: 2.5672x; 1.9474x over previous
import jax
import jax.numpy as jnp
from jax.experimental import pallas as pl
from jax.experimental.pallas import tpu as pltpu

T = 64
D = 1024
F = 1408
E = 16
ND = 2
DT = D // ND
SCALE = 1.0 / (2.0 ** 0.5)


def _dot(a, b, prec):
    return jax.lax.dot_general(a, b, (((1,), (0,)), ((), ())),
                               precision=prec,
                               preferred_element_type=jnp.float32)


def _moe_body(x_ref, rw_ref, eg_ref, eu_ref, ed_ref, sg_ref, su_ref, sd_ref,
              out_ref, comb_ref, gacc_ref, uacc_ref, sgacc_ref, suacc_ref):
    e = pl.program_id(0)
    d = pl.program_id(1)

    @pl.when((e == 0) & (d == 0))
    def _init():
        x = x_ref[...]
        lg = jax.lax.dot_general(x, rw_ref[...], (((1,), (1,)), ((), ())),
                                 precision=jax.lax.Precision.HIGHEST,
                                 preferred_element_type=jnp.float32)
        logits = lg[:, :E]
        iota = jax.lax.broadcasted_iota(jnp.int32, (T, E), 1)
        v1 = jnp.max(logits, axis=-1, keepdims=True)
        i1 = jnp.min(jnp.where(logits == v1, iota, E), axis=-1, keepdims=True)
        masked = jnp.where(iota == i1, -jnp.inf, logits)
        v2 = jnp.max(masked, axis=-1, keepdims=True)
        i2 = jnp.min(jnp.where(masked == v2, iota, E), axis=-1, keepdims=True)
        w1 = jax.nn.sigmoid(v1 - v2)
        w2 = jax.nn.sigmoid(v2 - v1)
        sscore = jax.nn.sigmoid(lg[:, E:E + 1])
        lane = jax.lax.broadcasted_iota(jnp.int32, (T, 128), 1)
        comb_full = (jnp.where(lane == i1, w1, 0.0)
                     + jnp.where(lane == i2, w2, 0.0)
                     + jnp.where(lane == E, sscore, 0.0)) * SCALE
        comb_ref[...] = comb_full
        out_ref[...] = jnp.zeros_like(out_ref)

    xx = x_ref[...]
    xd = jnp.where(d == 0, xx[:, :DT], xx[:, DT:])
    prec = jax.lax.Precision.DEFAULT

    pg = _dot(xd, eg_ref[0], prec)
    pu = _dot(xd, eu_ref[0], prec)

    @pl.when(d == 0)
    def _stash():
        gacc_ref[...] = pg
        uacc_ref[...] = pu

    @pl.when(d == 1)
    def _down():
        g = gacc_ref[...] + pg
        u = uacc_ref[...] + pu
        lane = jax.lax.broadcasted_iota(jnp.int32, (T, 128), 1)
        comb = comb_ref[...]
        w_e = jnp.sum(jnp.where(lane == e, comb, 0.0), axis=1, keepdims=True)
        h = jax.nn.silu(g) * u * w_e
        out_ref[...] += _dot(h, ed_ref[0], prec)

    @pl.when(e == 0)
    def _shared():
        psg = _dot(xd, sg_ref[0], prec)
        psu = _dot(xd, su_ref[0], prec)

        @pl.when(d == 0)
        def _sstash():
            sgacc_ref[...] = psg
            suacc_ref[...] = psu

        @pl.when(d == 1)
        def _sdown():
            g = sgacc_ref[...] + psg
            u = suacc_ref[...] + psu
            lane = jax.lax.broadcasted_iota(jnp.int32, (T, 128), 1)
            comb = comb_ref[...]
            w_s = jnp.sum(jnp.where(lane == E, comb, 0.0), axis=1,
                          keepdims=True)
            h = jax.nn.silu(g) * u * w_s
            out_ref[...] += _dot(h, sd_ref[0], prec)


def kernel(hidden_states, gate_w, shared_expert_gate_w, expert_gate_w,
           expert_up_w, expert_down_w, shared_gate_proj_w, shared_up_w,
           shared_down_w):
    x = hidden_states.reshape(T, D)
    rw = jnp.concatenate(
        [gate_w, shared_expert_gate_w,
         jnp.zeros((7, D), dtype=gate_w.dtype)], axis=0)

    out = pl.pallas_call(
        _moe_body,
        grid=(E, ND),
        in_specs=[
            pl.BlockSpec((T, D), lambda e, d: (0, 0)),
            pl.BlockSpec((24, D), lambda e, d: (0, 0)),
            pl.BlockSpec((1, DT, F), lambda e, d: (e, d, 0)),
            pl.BlockSpec((1, DT, F), lambda e, d: (e, d, 0)),
            pl.BlockSpec((1, F, D), lambda e, d: (e, 0, 0)),
            pl.BlockSpec((1, DT, F),
                         lambda e, d: (0, jnp.where(e == 0, d, 1), 0)),
            pl.BlockSpec((1, DT, F),
                         lambda e, d: (0, jnp.where(e == 0, d, 1), 0)),
            pl.BlockSpec((1, F, D), lambda e, d: (0, 0, 0)),
        ],
        out_specs=pl.BlockSpec((T, D), lambda e, d: (0, 0)),
        out_shape=jax.ShapeDtypeStruct((T, D), jnp.float32),
        scratch_shapes=[
            pltpu.VMEM((T, 128), jnp.float32),
            pltpu.VMEM((T, F), jnp.float32),
            pltpu.VMEM((T, F), jnp.float32),
            pltpu.VMEM((T, F), jnp.float32),
            pltpu.VMEM((T, F), jnp.float32),
        ],
        compiler_params=pltpu.CompilerParams(
            dimension_semantics=("arbitrary", "arbitrary"),
        ),
    )(x, rw, expert_gate_w, expert_up_w, expert_down_w,
      shared_gate_proj_w, shared_up_w, shared_down_w)
    return out


# speedup vs baseline: 2.6317x; 1.0251x over previous
import jax
import jax.numpy as jnp
from jax.experimental import pallas as pl
from jax.experimental.pallas import tpu as pltpu

T = 64
D = 1024
F = 1408
E = 16
ND = 2
DT = D // ND
SCALE = 1.0 / (2.0 ** 0.5)


def _dot(a, b, prec):
    return jax.lax.dot_general(a, b, (((1,), (0,)), ((), ())),
                               precision=prec,
                               preferred_element_type=jnp.float32)


def _moe_body(x_ref, rw_ref, eg_ref, eu_ref, ed_ref, sg_ref, su_ref, sd_ref,
              out_ref, comb_ref, gacc_ref, uacc_ref, sgacc_ref, suacc_ref):
    e = pl.program_id(0)
    d = pl.program_id(1)

    @pl.when((e == 0) & (d == 0))
    def _init():
        x = x_ref[...]
        lg = jax.lax.dot_general(x, rw_ref[...], (((1,), (1,)), ((), ())),
                                 precision=jax.lax.Precision.DEFAULT,
                                 preferred_element_type=jnp.float32)
        logits = lg[:, :E]
        iota = jax.lax.broadcasted_iota(jnp.int32, (T, E), 1)
        v1 = jnp.max(logits, axis=-1, keepdims=True)
        i1 = jnp.min(jnp.where(logits == v1, iota, E), axis=-1, keepdims=True)
        masked = jnp.where(iota == i1, -jnp.inf, logits)
        v2 = jnp.max(masked, axis=-1, keepdims=True)
        i2 = jnp.min(jnp.where(masked == v2, iota, E), axis=-1, keepdims=True)
        w1 = jax.nn.sigmoid(v1 - v2)
        w2 = jax.nn.sigmoid(v2 - v1)
        sscore = jax.nn.sigmoid(lg[:, E:E + 1])
        lane = jax.lax.broadcasted_iota(jnp.int32, (T, 128), 1)
        comb_full = (jnp.where(lane == i1, w1, 0.0)
                     + jnp.where(lane == i2, w2, 0.0)
                     + jnp.where(lane == E, sscore, 0.0)) * SCALE
        comb_ref[...] = comb_full
        out_ref[...] = jnp.zeros_like(out_ref)

    xx = x_ref[...]
    xd = jnp.where(d == 0, xx[:, :DT], xx[:, DT:])
    prec = jax.lax.Precision.DEFAULT

    pg = _dot(xd, eg_ref[0], prec)
    pu = _dot(xd, eu_ref[0], prec)

    @pl.when(d == 0)
    def _stash():
        gacc_ref[...] = pg
        uacc_ref[...] = pu

    @pl.when(d == 1)
    def _down():
        g = gacc_ref[...] + pg
        u = uacc_ref[...] + pu
        lane = jax.lax.broadcasted_iota(jnp.int32, (T, 128), 1)
        comb = comb_ref[...]
        w_e = jnp.sum(jnp.where(lane == e, comb, 0.0), axis=1, keepdims=True)
        h = jax.nn.silu(g) * u * w_e
        out_ref[...] += _dot(h, ed_ref[0], prec)

    @pl.when(e == 0)
    def _shared():
        psg = _dot(xd, sg_ref[0], prec)
        psu = _dot(xd, su_ref[0], prec)

        @pl.when(d == 0)
        def _sstash():
            sgacc_ref[...] = psg
            suacc_ref[...] = psu

        @pl.when(d == 1)
        def _sdown():
            g = sgacc_ref[...] + psg
            u = suacc_ref[...] + psu
            lane = jax.lax.broadcasted_iota(jnp.int32, (T, 128), 1)
            comb = comb_ref[...]
            w_s = jnp.sum(jnp.where(lane == E, comb, 0.0), axis=1,
                          keepdims=True)
            h = jax.nn.silu(g) * u * w_s
            out_ref[...] += _dot(h, sd_ref[0], prec)


def kernel(hidden_states, gate_w, shared_expert_gate_w, expert_gate_w,
           expert_up_w, expert_down_w, shared_gate_proj_w, shared_up_w,
           shared_down_w):
    x = hidden_states.reshape(T, D)
    rw = jnp.concatenate(
        [gate_w, shared_expert_gate_w,
         jnp.zeros((7, D), dtype=gate_w.dtype)], axis=0)

    out = pl.pallas_call(
        _moe_body,
        grid=(E, ND),
        in_specs=[
            pl.BlockSpec((T, D), lambda e, d: (0, 0)),
            pl.BlockSpec((24, D), lambda e, d: (0, 0)),
            pl.BlockSpec((1, DT, F), lambda e, d: (e, d, 0)),
            pl.BlockSpec((1, DT, F), lambda e, d: (e, d, 0)),
            pl.BlockSpec((1, F, D), lambda e, d: (e, 0, 0)),
            pl.BlockSpec((1, DT, F),
                         lambda e, d: (0, jnp.where(e == 0, d, 1), 0)),
            pl.BlockSpec((1, DT, F),
                         lambda e, d: (0, jnp.where(e == 0, d, 1), 0)),
            pl.BlockSpec((1, F, D), lambda e, d: (0, 0, 0)),
        ],
        out_specs=pl.BlockSpec((T, D), lambda e, d: (0, 0)),
        out_shape=jax.ShapeDtypeStruct((T, D), jnp.float32),
        scratch_shapes=[
            pltpu.VMEM((T, 128), jnp.float32),
            pltpu.VMEM((T, F), jnp.float32),
            pltpu.VMEM((T, F), jnp.float32),
            pltpu.VMEM((T, F), jnp.float32),
            pltpu.VMEM((T, F), jnp.float32),
        ],
        compiler_params=pltpu.CompilerParams(
            dimension_semantics=("arbitrary", "arbitrary"),
        ),
    )(x, rw, expert_gate_w, expert_up_w, expert_down_w,
      shared_gate_proj_w, shared_up_w, shared_down_w)
    return out
